# Initial kernel scaffold; baseline (speedup 1.0000x reference)
import jax
import jax.numpy as jnp
from jax.experimental import pallas as pl


def kernel(ds_in, ds_out, layer_edge_index):
    raise NotImplementedError("write your pallas kernel here")


# speedup vs baseline: 6.3320x; 6.3320x over previous
import functools

import jax
import jax.numpy as jnp
from jax import lax
from jax.experimental import pallas as pl
from jax.experimental.pallas import tpu as pltpu
from jax.experimental.pallas import tpu_sc as plsc

_NC = 2
_NS = 16
_G = 80
_ZR = 32


def _sc_accumulate(src, recv, ds_out):
    e_total = src.shape[0]
    n, d = ds_out.shape
    nw = _NC * _NS
    e_per_tile = e_total // nw
    n_chunks = e_per_tile // _G
    n_pad = ((n + _NS * _ZR - 1) // (_NS * _ZR)) * (_NS * _ZR)
    rows_per_tile = n_pad // _NS
    n_zero = rows_per_tile // _ZR

    mesh = plsc.VectorSubcoreMesh(core_axis_name="c", subcore_axis_name="s")

    @functools.partial(
        pl.kernel,
        out_type=(
            jax.ShapeDtypeStruct((_NC, n_pad, d), jnp.float32),
            jax.ShapeDtypeStruct((nw * n_pad,), jnp.float32),
        ),
        mesh=mesh,
        compiler_params=pltpu.CompilerParams(needs_layout_passes=False),
        scratch_types=[
            pltpu.VMEM((_G,), jnp.int32),
            pltpu.VMEM((_G,), jnp.int32),
            pltpu.VMEM((_G, d), jnp.float32),
            pltpu.VMEM((_ZR, d), jnp.float32),
            pltpu.VMEM((n_pad,), jnp.float32),
            pltpu.VMEM_SHARED((n_pad, d), jnp.float32),
        ],
    )
    def accumulate(src_hbm, recv_hbm, dsout_hbm, pmsg_hbm, cnt_hbm,
                   sidx, ridx, rows, zb, lhist, msg_sh):
        c = lax.axis_index("c")
        s = lax.axis_index("s")
        wid = s * _NC + c

        z16 = jnp.zeros((16,), jnp.float32)
        o16 = jnp.ones((16,), jnp.float32)

        def fill_z(i, carry):
            for j in range(d // 16):
                zb[i, pl.ds(j * 16, 16)] = z16
            return carry

        lax.fori_loop(0, _ZR, fill_z, 0)

        def fill_h(i, carry):
            lhist[pl.ds(i * 16, 16)] = z16
            return carry

        lax.fori_loop(0, n_pad // 16, fill_h, 0)

        row0 = s * rows_per_tile

        def zero_step(k, carry):
            pltpu.sync_copy(zb, msg_sh.at[pl.ds(row0 + k * _ZR, _ZR)])
            return carry

        lax.fori_loop(0, n_zero, zero_step, 0)
        plsc.subcore_barrier()

        ebase = wid * e_per_tile

        def edge_step(i, carry):
            off = ebase + i * _G
            pltpu.sync_copy(src_hbm.at[pl.ds(off, _G)], sidx)
            pltpu.sync_copy(recv_hbm.at[pl.ds(off, _G)], ridx)
            pltpu.sync_copy(dsout_hbm.at[sidx], rows)
            pltpu.sync_copy(rows, msg_sh.at[ridx], add=True)
            for j in range(_G // 16):
                idx16 = ridx[pl.ds(j * 16, 16)]
                plsc.addupdate_scatter(lhist, [idx16], o16)
            return carry

        lax.fori_loop(0, n_chunks, edge_step, 0)
        plsc.subcore_barrier()

        pltpu.sync_copy(msg_sh.at[pl.ds(row0, rows_per_tile)],
                        pmsg_hbm.at[c, pl.ds(row0, rows_per_tile)])
        pltpu.sync_copy(lhist, cnt_hbm.at[pl.ds(wid * n_pad, n_pad)])

    pmsg, cnt = accumulate(src, recv, ds_out)
    return pmsg, cnt.reshape(nw, n_pad)


def _combine_body(di_ref, pm_ref, pc_ref, o_ref):
    m = di_ref[...] + pm_ref[0] + pm_ref[1]
    den = 1.0 + jnp.sum(pc_ref[...], axis=1, keepdims=True)
    o_ref[...] = m / den


def _combine(ds_in, pmsg, pcnt_t):
    n, d = ds_in.shape
    br = 1000
    nw = pcnt_t.shape[1]
    return pl.pallas_call(
        _combine_body,
        grid=(n // br,),
        in_specs=[
            pl.BlockSpec((br, d), lambda i: (i, 0)),
            pl.BlockSpec((_NC, br, d), lambda i: (0, i, 0)),
            pl.BlockSpec((br, nw), lambda i: (i, 0)),
        ],
        out_specs=pl.BlockSpec((br, d), lambda i: (i, 0)),
        out_shape=jax.ShapeDtypeStruct((n, d), jnp.float32),
    )(ds_in, pmsg, pcnt_t)


def kernel(ds_in, ds_out, layer_edge_index):
    lei = layer_edge_index.astype(jnp.int32)
    pmsg, cnt = _sc_accumulate(lei[0], lei[1], ds_out)
    return _combine(ds_in, pmsg, cnt.T)


# speedup vs baseline: 11.1984x; 1.7685x over previous
import functools

import jax
import jax.numpy as jnp
from jax import lax
from jax.experimental import pallas as pl
from jax.experimental.pallas import tpu as pltpu
from jax.experimental.pallas import tpu_sc as plsc

_NC = 2
_NS = 16
_G = 80
_ZR = 32
_NB = 3


def _sc_accumulate(src, recv, ds_out):
    e_total = src.shape[0]
    n, d = ds_out.shape
    nw = _NC * _NS
    e_per_tile = e_total // nw
    n_chunks = e_per_tile // _G
    n_pad = ((n + _NS * _ZR - 1) // (_NS * _ZR)) * (_NS * _ZR)
    rows_per_tile = n_pad // _NS
    n_zero = rows_per_tile // _ZR

    mesh = plsc.VectorSubcoreMesh(core_axis_name="c", subcore_axis_name="s")

    @functools.partial(
        pl.kernel,
        out_type=(
            jax.ShapeDtypeStruct((_NC, n_pad, d), jnp.float32),
            jax.ShapeDtypeStruct((nw * n_pad,), jnp.float32),
        ),
        mesh=mesh,
        compiler_params=pltpu.CompilerParams(needs_layout_passes=False),
        scratch_types=[
            pltpu.VMEM((_NB, _G), jnp.int32),
            pltpu.VMEM((_NB, _G), jnp.int32),
            pltpu.VMEM((_NB, _G, d), jnp.float32),
            pltpu.VMEM((_ZR, d), jnp.float32),
            pltpu.VMEM((n_pad,), jnp.float32),
            pltpu.VMEM_SHARED((n_pad, d), jnp.float32),
            pltpu.SemaphoreType.DMA((_NB,)),
            pltpu.SemaphoreType.DMA((_NB,)),
            pltpu.SemaphoreType.DMA((_NB,)),
        ],
    )
    def accumulate(src_hbm, recv_hbm, dsout_hbm, pmsg_hbm, cnt_hbm,
                   sidx, ridx, rows, zb, lhist, msg_sh, isem, gsem, ssem):
        c = lax.axis_index("c")
        s = lax.axis_index("s")
        wid = s * _NC + c

        z16 = jnp.zeros((16,), jnp.float32)
        o16 = jnp.ones((16,), jnp.float32)

        def fill_z(i, carry):
            for j in range(d // 16):
                zb[i, pl.ds(j * 16, 16)] = z16
            return carry

        lax.fori_loop(0, _ZR, fill_z, 0)

        def fill_h(i, carry):
            lhist[pl.ds(i * 16, 16)] = z16
            return carry

        lax.fori_loop(0, n_pad // 16, fill_h, 0)

        row0 = s * rows_per_tile

        def zero_step(k, carry):
            pltpu.sync_copy(zb, msg_sh.at[pl.ds(row0 + k * _ZR, _ZR)])
            return carry

        lax.fori_loop(0, n_zero, zero_step, 0)
        plsc.subcore_barrier()

        ebase = wid * e_per_tile

        def start_idx(chunk, slot):
            off = ebase + chunk * _G
            pltpu.async_copy(src_hbm.at[pl.ds(off, _G)], sidx.at[slot],
                             isem.at[slot])
            pltpu.async_copy(recv_hbm.at[pl.ds(off, _G)], ridx.at[slot],
                             isem.at[slot])

        start_idx(0, 0)
        start_idx(1, 1)

        def edge_step(i, carry):
            b = lax.rem(i, _NB)
            b1 = lax.rem(i + 2, _NB)
            off = ebase + i * _G
            pltpu.make_async_copy(src_hbm.at[pl.ds(off, _G)], sidx.at[b],
                                  isem.at[b]).wait()
            pltpu.make_async_copy(recv_hbm.at[pl.ds(off, _G)], ridx.at[b],
                                  isem.at[b]).wait()
            pltpu.async_copy(dsout_hbm.at[sidx.at[b]], rows.at[b], gsem.at[b])

            for j in range(_G // 16):
                idx16 = ridx[b, pl.ds(j * 16, 16)]
                plsc.addupdate_scatter(lhist, [idx16], o16)

            @pl.when(i >= 1)
            def _():
                pltpu.make_async_copy(rows.at[b1], msg_sh.at[ridx.at[b1]],
                                      ssem.at[b1]).wait()

            @pl.when(i + 2 < n_chunks)
            def _():
                start_idx(i + 2, b1)

            pltpu.make_async_copy(dsout_hbm.at[sidx.at[b]], rows.at[b],
                                  gsem.at[b]).wait()
            pltpu.async_copy(rows.at[b], msg_sh.at[ridx.at[b]], ssem.at[b],
                             add=True)
            return carry

        lax.fori_loop(0, n_chunks, edge_step, 0)

        b_last = (n_chunks - 1) % _NB
        pltpu.make_async_copy(rows.at[b_last], msg_sh.at[ridx.at[b_last]],
                              ssem.at[b_last]).wait()
        plsc.subcore_barrier()

        pltpu.sync_copy(msg_sh.at[pl.ds(row0, rows_per_tile)],
                        pmsg_hbm.at[c, pl.ds(row0, rows_per_tile)])
        pltpu.sync_copy(lhist, cnt_hbm.at[pl.ds(wid * n_pad, n_pad)])

    pmsg, cnt = accumulate(src, recv, ds_out)
    return pmsg, cnt.reshape(nw, n_pad)


def _combine_body(di_ref, pm_ref, pc_ref, o_ref):
    m = di_ref[...] + pm_ref[0] + pm_ref[1]
    den = 1.0 + jnp.sum(pc_ref[...], axis=1, keepdims=True)
    o_ref[...] = m / den


def _combine(ds_in, pmsg, pcnt_t):
    n, d = ds_in.shape
    br = 1000
    nw = pcnt_t.shape[1]
    return pl.pallas_call(
        _combine_body,
        grid=(n // br,),
        in_specs=[
            pl.BlockSpec((br, d), lambda i: (i, 0)),
            pl.BlockSpec((_NC, br, d), lambda i: (0, i, 0)),
            pl.BlockSpec((br, nw), lambda i: (i, 0)),
        ],
        out_specs=pl.BlockSpec((br, d), lambda i: (i, 0)),
        out_shape=jax.ShapeDtypeStruct((n, d), jnp.float32),
    )(ds_in, pmsg, pcnt_t)


def kernel(ds_in, ds_out, layer_edge_index):
    lei = layer_edge_index.astype(jnp.int32)
    pmsg, cnt = _sc_accumulate(lei[0], lei[1], ds_out)
    return _combine(ds_in, pmsg, cnt.T)
